# Initial kernel scaffold; baseline (speedup 1.0000x reference)
import jax
import jax.numpy as jnp
from jax.experimental import pallas as pl


def kernel(W_S, W_C, W_F, W_R, W_I, U, fc1_w, fc1_b, fc2_w, fc2_b, delay_chain, eligibility_trace, C, stimuli_seq, noise_seq):
    raise NotImplementedError("write your pallas kernel here")


# speedup vs baseline: 1.2823x; 1.2823x over previous
import functools

import jax
import jax.numpy as jnp
from jax import lax
from jax.experimental import pallas as pl
from jax.experimental.pallas import tpu as pltpu


def _phase_a_kernel(
    steps_ref,
    prev_ref,
    init_ref,
    l_ref,
    lam_ref,
    w1_ref,
    w2_ref,
    w3_ref,
    b_ref,
    out_ref,
    ec_ref,
    dl_ref,
    el_ref,
    e_s,
    *,
    n_stim, n_resp, n_ro, n_delay, hidden,
    chunk, last_local, dt, beta, psi, phi, rho,
):
    f32 = jnp.float32
    S, R, RO, H = n_stim, n_resp, n_ro, hidden
    CH = chunk
    nd = n_delay
    cid = pl.program_id(0)

    stim = steps_ref[:, 0:S]
    noise = steps_ref[:, S:S + R]

    @pl.when(cid == 0)
    def _():
        for i in range(nd - 1):
            lo = (nd - 2 - i) * S
            e_s[i:i + 1, :] = init_ref[:, lo:lo + S]

    @pl.when(cid != 0)
    def _():
        e_s[0:nd - 1, :] = prev_ref[CH - (nd - 1):CH, 0:S]

    e_s[nd - 1:nd - 1 + CH, :] = stim
    d_mat = jnp.concatenate(
        [e_s[nd - 1 - d:nd - 1 - d + CH, :] for d in range(nd)],
        axis=1)

    elig = jnp.dot(l_ref[...], d_mat, preferred_element_type=f32)

    fc1_b = b_ref[:, 0:H]
    fc2_b = b_ref[:, H:H + RO]

    res1 = jnp.dot(stim, w1_ref[...], preferred_element_type=f32)
    ws_out = res1[:, 0:RO]
    wc_out = res1[:, RO:RO + R]
    h = jnp.maximum(res1[:, RO + R:RO + R + H] + fc1_b, 0.0)

    lhs2 = jnp.concatenate([h, elig], axis=1)
    res2 = jnp.dot(lhs2, w2_ref[...], preferred_element_type=f32)
    valence = res2[:, 0:RO] + fc2_b
    temporal = res2[:, RO:2 * RO]
    ro_pred = ws_out * valence

    lhs3 = jnp.concatenate([ro_pred, jnp.maximum(ro_pred, 0.0)], axis=1)
    res3 = jnp.dot(lhs3, w3_ref[...], preferred_element_type=f32)
    proactive = -res3[:, 0:R]
    reactive = jnp.maximum(res3[:, R:2 * R], 0.0)
    control = phi * (proactive + reactive)
    excitation = rho * wc_out

    bdt = beta * dt
    r_vec = bdt * (excitation + noise - 0.05 * control)
    s_vec = bdt * (excitation + control)

    out_ref[:, 0:2 * R] = jnp.zeros((CH, 2 * R), f32)
    out_ref[:, 2 * R:2 * R + RO] = ro_pred
    out_ref[:, 2 * R + RO:2 * R + 2 * RO] = temporal
    out_ref[:, 2 * R + 2 * RO:2 * R + 2 * RO + R] = r_vec
    out_ref[:, 2 * R + 2 * RO + R:2 * R + 2 * RO + 2 * R] = s_vec

    ec_ref[0, :, :] = elig[CH - 1:CH, :]
    dl_ref[0, :, :] = d_mat[last_local:last_local + 1, :]
    el_ref[0, :, :] = elig[last_local:last_local + 1, :]


def _phase_b_kernel(
    outa_ref,
    ec_ref,
    dl_ref,
    el_ref,
    init_ref,
    lam_ref,
    u2d_ref,
    wi_ref,
    out_ref,
    state_out_ref,
    carry_s,
    c_s,
    *,
    n_stim, n_resp, n_ro, n_delay,
    chunk, last_chunk, last_local, lam_chunk, lam_last,
    response_threshold,
):
    f32 = jnp.float32
    R, RO = n_resp, n_ro
    DS = n_delay * n_stim
    CH = chunk
    cid = pl.program_id(0)

    @pl.when(cid == 0)
    def _():
        carry_s[...] = init_ref[:, 0:DS]
        c_s[...] = init_ref[:, DS:DS + R]

    carry = carry_s[...]

    corr = jnp.dot(carry, u2d_ref[...], preferred_element_type=f32)
    out_ref[...] = outa_ref[...]
    out_ref[:, 2 * R + RO:2 * R + 2 * RO] = (
        outa_ref[:, 2 * R + RO:2 * R + 2 * RO] + lam_ref[...] * corr)

    w_i = wi_ref[...]
    rs_lo = 2 * R + 2 * RO

    def body(j, c):
        row = outa_ref[pl.ds(j, 1), rs_lo:rs_lo + 2 * R]
        r = row[:, 0:R]
        s = row[:, R:2 * R]
        inh = jnp.dot(c, w_i, preferred_element_type=f32)
        c_new = jax.nn.sigmoid(c + r - s * c - (c + 0.05) * inh)
        out_ref[pl.ds(j, 1), 0:R] = c_new
        return c_new

    c_final = lax.fori_loop(0, CH, body, c_s[...])
    c_s[...] = c_final

    c_all = out_ref[:, 0:R]
    out_ref[:, R:2 * R] = (c_all > response_threshold).astype(f32)
    out_ref[:, 2 * R + 2 * RO:2 * R + 2 * RO + 2 * R] = jnp.zeros((CH, 2 * R), f32)

    @pl.when(cid == last_chunk)
    def _():
        state_out_ref[:, 0:DS] = dl_ref[0, :, :]
        state_out_ref[:, DS:2 * DS] = el_ref[0, :, :] + lam_last * carry
        state_out_ref[:, 2 * DS:2 * DS + R] = out_ref[last_local:last_local + 1,
                                                      0:R]

    carry_s[...] = ec_ref[0, :, :] + lam_chunk * carry


def _build_slabs(W_S, W_C, W_F, W_R, U, fc1_w, fc1_b, fc2_w, fc2_b,
                 n_stim, n_resp, n_ro, n_delay, hidden):
    f32 = jnp.float32
    S, R, RO, H = n_stim, n_resp, n_ro, hidden
    DS = n_delay * S

    w1 = jnp.zeros((S, RO + R + H), f32)
    w1 = w1.at[:, :RO].set(W_S.T.astype(f32))
    w1 = w1.at[:, RO:RO + R].set(W_C.T.astype(f32))
    w1 = w1.at[:, RO + R:].set(fc1_w.T.astype(f32))

    u2d = U.astype(f32).reshape(RO, DS).T
    w2 = jnp.zeros((H + DS, 2 * RO), f32)
    w2 = w2.at[:H, :RO].set(fc2_w.T.astype(f32))
    w2 = w2.at[H:H + DS, RO:].set(u2d)

    w3 = jnp.zeros((2 * RO, 2 * R), f32)
    w3 = w3.at[:RO, :R].set(W_F.astype(f32))
    w3 = w3.at[RO:2 * RO, R:2 * R].set(W_R.T.astype(f32))

    bias = jnp.concatenate([fc1_b.astype(f32),
                            fc2_b.astype(f32)]).reshape(1, H + RO)
    return w1, w2, w3, u2d, bias


def kernel(W_S, W_C, W_F, W_R, W_I, U, fc1_w, fc1_b, fc2_w, fc2_b,
           delay_chain, eligibility_trace, C, stimuli_seq, noise_seq):
    f32 = jnp.float32
    hp = dict(dt=0.1, beta=0.1, lambda_decay=0.95, psi=0.1, phi=0.1, rho=0.1,
              response_threshold=0.5)
    n_ro, n_stim = W_S.shape
    n_resp = W_C.shape[0]
    n_delay = delay_chain.shape[0]
    hidden = fc1_b.shape[0]
    S, R, RO, H = n_stim, n_resp, n_ro, hidden
    DS = n_delay * S
    T = int(stimuli_seq.shape[0])
    OUT_W = 128
    assert 2 * R + 2 * RO + 2 * R <= OUT_W

    max_chunk = 256
    chunk = min(max_chunk, ((T + 7) // 8) * 8)
    t_pad = ((T + chunk - 1) // chunk) * chunk
    num_chunks = t_pad // chunk
    last_chunk = (T - 1) // chunk
    last_local = (T - 1) % chunk

    lam = float(hp["lambda_decay"])
    idx = jnp.arange(chunk)
    diff = idx[:, None] - idx[None, :]
    lam_mat = jnp.where(
        diff >= 0,
        jnp.power(jnp.float32(lam), jnp.maximum(diff, 0).astype(f32)),
        0.0).astype(f32)
    lam_pows = jnp.power(jnp.float32(lam),
                         (idx + 1).astype(f32)).reshape(chunk, 1)

    w1, w2, w3, u2d, bias = _build_slabs(
        W_S, W_C, W_F, W_R, U, fc1_w, fc1_b, fc2_w, fc2_b,
        S, R, RO, n_delay, H)

    stim = jnp.zeros((t_pad, S), f32).at[:T].set(
        stimuli_seq.reshape(T, S).astype(f32))
    noz = jnp.zeros((t_pad, R), f32).at[:T].set(
        noise_seq.reshape(T, R).astype(f32))
    steps = jnp.concatenate([stim, noz], axis=1)

    init_delay = delay_chain.astype(f32).reshape(1, DS)
    init_ec = jnp.concatenate(
        [eligibility_trace.astype(f32).reshape(1, DS),
         C.astype(f32).reshape(1, R)], axis=1)

    bdt = float(hp["beta"]) * float(hp["dt"])
    wi_scaled = (bdt * float(hp["psi"])) * W_I.astype(f32)

    a_fn = functools.partial(
        _phase_a_kernel,
        n_stim=S, n_resp=R, n_ro=RO, n_delay=n_delay, hidden=H,
        chunk=chunk, last_local=last_local,
        dt=float(hp["dt"]), beta=float(hp["beta"]),
        psi=float(hp["psi"]), phi=float(hp["phi"]), rho=float(hp["rho"]),
    )

    def const_spec(shape):
        return pl.BlockSpec(shape, lambda c: (0,) * len(shape))

    outa, ec, dl, el = pl.pallas_call(
        a_fn,
        grid=(num_chunks,),
        in_specs=[
            pl.BlockSpec((chunk, S + R), lambda c: (c, 0)),
            pl.BlockSpec((chunk, S + R), lambda c: (jnp.maximum(c - 1, 0), 0)),
            const_spec((1, DS)),
            const_spec((chunk, chunk)),
            const_spec((chunk, 1)),
            const_spec(tuple(w1.shape)),
            const_spec(tuple(w2.shape)),
            const_spec(tuple(w3.shape)),
            const_spec(tuple(bias.shape)),
        ],
        out_specs=(
            pl.BlockSpec((chunk, OUT_W), lambda c: (c, 0)),
            pl.BlockSpec((1, 1, DS), lambda c: (c, 0, 0)),
            pl.BlockSpec((1, 1, DS), lambda c: (c, 0, 0)),
            pl.BlockSpec((1, 1, DS), lambda c: (c, 0, 0)),
        ),
        out_shape=(
            jax.ShapeDtypeStruct((t_pad, OUT_W), f32),
            jax.ShapeDtypeStruct((num_chunks, 1, DS), f32),
            jax.ShapeDtypeStruct((num_chunks, 1, DS), f32),
            jax.ShapeDtypeStruct((num_chunks, 1, DS), f32),
        ),
        scratch_shapes=[
            pltpu.VMEM((chunk + n_delay - 1, S), f32),
        ],
        compiler_params=pltpu.CompilerParams(
            dimension_semantics=("arbitrary",)),
    )(steps, steps, init_delay, lam_mat, lam_pows, w1, w2, w3, bias)

    b_fn = functools.partial(
        _phase_b_kernel,
        n_stim=S, n_resp=R, n_ro=RO, n_delay=n_delay,
        chunk=chunk, last_chunk=last_chunk, last_local=last_local,
        lam_chunk=float(lam ** chunk), lam_last=float(lam ** (last_local + 1)),
        response_threshold=float(hp["response_threshold"]),
    )

    per_step_out, final_state = pl.pallas_call(
        b_fn,
        grid=(num_chunks,),
        in_specs=[
            pl.BlockSpec((chunk, OUT_W), lambda c: (c, 0)),
            pl.BlockSpec((1, 1, DS), lambda c: (c, 0, 0)),
            pl.BlockSpec((1, 1, DS), lambda c: (c, 0, 0)),
            pl.BlockSpec((1, 1, DS), lambda c: (c, 0, 0)),
            const_spec((1, DS + R)),
            const_spec((chunk, 1)),
            const_spec((DS, RO)),
            const_spec((R, R)),
        ],
        out_specs=(
            pl.BlockSpec((chunk, OUT_W), lambda c: (c, 0)),
            const_spec((1, 2 * DS + R)),
        ),
        out_shape=(
            jax.ShapeDtypeStruct((t_pad, OUT_W), f32),
            jax.ShapeDtypeStruct((1, 2 * DS + R), f32),
        ),
        scratch_shapes=[
            pltpu.VMEM((1, DS), f32),
            pltpu.VMEM((1, R), f32),
        ],
        input_output_aliases={0: 0},
        compiler_params=pltpu.CompilerParams(
            dimension_semantics=("arbitrary",)),
    )(outa, ec, dl, el, init_ec, lam_pows, u2d, wi_scaled)

    rows = per_step_out[:T]
    resp = rows[:, 0:R]
    disc = rows[:, R:2 * R]
    ro_pred = rows[:, 2 * R:2 * R + RO]
    temporal = rows[:, 2 * R + RO:2 * R + 2 * RO]
    fs = final_state[0]
    new_state = dict(
        delay_chain=fs[0:DS].reshape(n_delay, S),
        eligibility_trace=fs[DS:2 * DS].reshape(n_delay, S),
        C=fs[2 * DS:2 * DS + R],
    )
    return resp, ro_pred, temporal, disc, new_state


# speedup vs baseline: 11.4250x; 8.9098x over previous
import functools

import jax
import jax.numpy as jnp
from jax import lax
from jax.experimental import pallas as pl
from jax.experimental.pallas import tpu as pltpu

_NSWEEP = 14


def _pro_chunk_kernel(
    steps_ref,
    init_ref,
    l_ref,
    lam_ref,
    w1_ref,
    w2_ref,
    u2d_ref,
    w3_ref,
    wi_ref,
    b_ref,
    out_ref,
    state_out_ref,
    state_s,
    e_s,
    c_s,
    *,
    n_stim, n_resp, n_ro, n_delay, hidden,
    chunk, last_chunk, last_local,
    dt, beta, psi, phi, rho, response_threshold,
):
    f32 = jnp.float32
    bf16 = jnp.bfloat16
    S, R, RO, H = n_stim, n_resp, n_ro, hidden
    DS = n_delay * S
    CH = chunk
    cid = pl.program_id(0)

    @pl.when(cid == 0)
    def _():
        state_s[...] = init_ref[...]

    stim = steps_ref[:, 0:S]
    noise = steps_ref[:, S:S + R]

    for i in range(n_delay - 1):
        lo = (n_delay - 2 - i) * S
        e_s[i:i + 1, :] = state_s[:, lo:lo + S]
    e_s[n_delay - 1:n_delay - 1 + CH, :] = stim
    d_mat = jnp.concatenate(
        [e_s[n_delay - 1 - d:n_delay - 1 - d + CH, :] for d in range(n_delay)],
        axis=1)

    elig = (jnp.dot(l_ref[...], d_mat.astype(bf16),
                    preferred_element_type=f32)
            + lam_ref[...] * state_s[:, DS:2 * DS])

    fc1_b = b_ref[:, 0:H]
    fc2_b = b_ref[:, H:H + RO]

    res1 = jnp.dot(stim, w1_ref[...], preferred_element_type=f32)
    ws_out = res1[:, 0:RO]
    wc_out = res1[:, RO:RO + R]
    h = jnp.maximum(res1[:, RO + R:RO + R + H] + fc1_b, 0.0)

    valence = jnp.dot(h, w2_ref[...], preferred_element_type=f32) + fc2_b
    temporal = jnp.dot(elig.astype(bf16), u2d_ref[...],
                       preferred_element_type=f32)
    ro_pred = ws_out * valence

    lhs3 = jnp.concatenate([ro_pred, jnp.maximum(ro_pred, 0.0)], axis=1)
    res3 = jnp.dot(lhs3, w3_ref[...], preferred_element_type=f32)
    proactive = -res3[:, 0:R]
    reactive = jnp.maximum(res3[:, R:2 * R], 0.0)
    control = phi * (proactive + reactive)
    excitation = rho * wc_out

    bdt = beta * dt
    r_vec = bdt * (excitation + noise - 0.05 * control)
    s_vec = bdt * (excitation + control)

    c0 = state_s[:, 2 * DS:2 * DS + R]
    c_s[0:1, :] = c0
    c_s[1:CH + 1, :] = jnp.broadcast_to(c0, (CH, R))
    w_i = wi_ref[...]
    for _ in range(_NSWEEP):
        src = c_s[0:CH, :]
        inh = jnp.dot(src, w_i, preferred_element_type=f32)
        pre = src + r_vec - s_vec * src - (src + 0.05) * inh
        c_s[1:CH + 1, :] = jax.nn.sigmoid(pre)

    c_all = c_s[1:CH + 1, :]

    out_ref[...] = jnp.zeros(out_ref.shape, f32)
    out_ref[:, 0:R] = c_all
    out_ref[:, R:2 * R] = (c_all > response_threshold).astype(f32)
    out_ref[:, 2 * R:2 * R + RO] = ro_pred
    out_ref[:, 2 * R + RO:2 * R + 2 * RO] = temporal

    state_s[:, 0:DS] = d_mat[CH - 1:CH, :]
    state_s[:, DS:2 * DS] = elig[CH - 1:CH, :]
    state_s[:, 2 * DS:2 * DS + R] = c_s[CH:CH + 1, :]

    @pl.when(cid == last_chunk)
    def _():
        state_out_ref[:, 0:DS] = d_mat[last_local:last_local + 1, :]
        state_out_ref[:, DS:2 * DS] = elig[last_local:last_local + 1, :]
        state_out_ref[:, 2 * DS:2 * DS + R] = c_s[last_local + 1:last_local + 2, :]


def kernel(W_S, W_C, W_F, W_R, W_I, U, fc1_w, fc1_b, fc2_w, fc2_b,
           delay_chain, eligibility_trace, C, stimuli_seq, noise_seq):
    f32 = jnp.float32
    bf16 = jnp.bfloat16
    hp = dict(dt=0.1, beta=0.1, lambda_decay=0.95, psi=0.1, phi=0.1, rho=0.1,
              response_threshold=0.5)
    n_ro, n_stim = W_S.shape
    n_resp = W_C.shape[0]
    n_delay = delay_chain.shape[0]
    hidden = fc1_b.shape[0]
    S, R, RO, H = n_stim, n_resp, n_ro, hidden
    DS = n_delay * S
    state_len = 2 * DS + R
    T = int(stimuli_seq.shape[0])
    OUT_W = 128
    assert 2 * R + 2 * RO <= OUT_W

    max_chunk = 256
    chunk = min(max_chunk, ((T + 7) // 8) * 8)
    t_pad = ((T + chunk - 1) // chunk) * chunk
    num_chunks = t_pad // chunk
    last_chunk = (T - 1) // chunk
    last_local = (T - 1) % chunk

    lam = float(hp["lambda_decay"])
    idx = jnp.arange(chunk)
    diff = idx[:, None] - idx[None, :]
    lam_mat = jnp.where(
        diff >= 0,
        jnp.power(jnp.float32(lam), jnp.maximum(diff, 0).astype(f32)),
        0.0).astype(bf16)
    lam_pows = jnp.power(jnp.float32(lam),
                         (idx + 1).astype(f32)).reshape(chunk, 1)

    w1 = jnp.zeros((S, RO + R + H), f32)
    w1 = w1.at[:, :RO].set(W_S.T.astype(f32))
    w1 = w1.at[:, RO:RO + R].set(W_C.T.astype(f32))
    w1 = w1.at[:, RO + R:].set(fc1_w.T.astype(f32))

    w2 = fc2_w.T.astype(f32)
    u2d = U.astype(f32).reshape(RO, DS).T.astype(bf16)

    w3 = jnp.zeros((2 * RO, 2 * R), f32)
    w3 = w3.at[:RO, :R].set(W_F.astype(f32))
    w3 = w3.at[RO:2 * RO, R:2 * R].set(W_R.T.astype(f32))

    bdt = float(hp["beta"]) * float(hp["dt"])
    w_i = (bdt * float(hp["psi"])) * W_I.astype(f32)
    bias = jnp.concatenate([fc1_b.astype(f32),
                            fc2_b.astype(f32)]).reshape(1, H + RO)

    stim = jnp.zeros((t_pad, S), f32).at[:T].set(
        stimuli_seq.reshape(T, S).astype(f32))
    noz = jnp.zeros((t_pad, R), f32).at[:T].set(
        noise_seq.reshape(T, R).astype(f32))
    steps = jnp.concatenate([stim, noz], axis=1)

    init_state = jnp.concatenate(
        [delay_chain.astype(f32).reshape(1, DS),
         eligibility_trace.astype(f32).reshape(1, DS),
         C.astype(f32).reshape(1, R)], axis=1)

    kernel_fn = functools.partial(
        _pro_chunk_kernel,
        n_stim=S, n_resp=R, n_ro=RO, n_delay=n_delay, hidden=H,
        chunk=chunk, last_chunk=last_chunk, last_local=last_local,
        dt=float(hp["dt"]), beta=float(hp["beta"]),
        psi=float(hp["psi"]), phi=float(hp["phi"]), rho=float(hp["rho"]),
        response_threshold=float(hp["response_threshold"]),
    )

    def const_spec(shape):
        return pl.BlockSpec(shape, lambda c: (0,) * len(shape))

    per_step_out, final_state = pl.pallas_call(
        kernel_fn,
        grid=(num_chunks,),
        in_specs=[
            pl.BlockSpec((chunk, S + R), lambda c: (c, 0)),
            const_spec((1, state_len)),
            const_spec((chunk, chunk)),
            const_spec((chunk, 1)),
            const_spec(tuple(w1.shape)),
            const_spec(tuple(w2.shape)),
            const_spec(tuple(u2d.shape)),
            const_spec(tuple(w3.shape)),
            const_spec(tuple(w_i.shape)),
            const_spec(tuple(bias.shape)),
        ],
        out_specs=(
            pl.BlockSpec((chunk, OUT_W), lambda c: (c, 0)),
            const_spec((1, state_len)),
        ),
        out_shape=(
            jax.ShapeDtypeStruct((t_pad, OUT_W), f32),
            jax.ShapeDtypeStruct((1, state_len), f32),
        ),
        scratch_shapes=[
            pltpu.VMEM((1, state_len), f32),
            pltpu.VMEM((chunk + n_delay - 1, S), f32),
            pltpu.VMEM((chunk + 8, R), f32),
        ],
        compiler_params=pltpu.CompilerParams(
            dimension_semantics=("arbitrary",)),
    )(steps, init_state, lam_mat, lam_pows, w1, w2, u2d, w3, w_i, bias)

    rows = per_step_out[:T]
    resp = rows[:, 0:R]
    disc = rows[:, R:2 * R]
    ro_pred = rows[:, 2 * R:2 * R + RO]
    temporal = rows[:, 2 * R + RO:2 * R + 2 * RO]
    fs = final_state[0]
    new_state = dict(
        delay_chain=fs[0:DS].reshape(n_delay, S),
        eligibility_trace=fs[DS:2 * DS].reshape(n_delay, S),
        C=fs[2 * DS:2 * DS + R],
    )
    return resp, ro_pred, temporal, disc, new_state


# speedup vs baseline: 15.0888x; 1.3207x over previous
import functools

import jax
import jax.numpy as jnp
from jax import lax
from jax.experimental import pallas as pl
from jax.experimental.pallas import tpu as pltpu

_NSWEEP = 20
_LAG = 7


def _pro_chunk_kernel(
    steps_ref,
    init_ref,
    l_ref,
    lrows_ref,
    lam_ref,
    w1_ref,
    w2_ref,
    u2d_ref,
    w3_ref,
    wi_ref,
    b_ref,
    out_ref,
    state_out_ref,
    state_s,
    e_s,
    eb_s,
    c_s,
    *,
    n_stim, n_resp, n_ro, n_delay, hidden,
    chunk, last_chunk, last_local,
    dt, beta, psi, phi, rho, response_threshold,
):
    f32 = jnp.float32
    bf16 = jnp.bfloat16
    S, R, RO, H = n_stim, n_resp, n_ro, hidden
    DS = n_delay * S
    CH = chunk
    cid = pl.program_id(0)

    @pl.when(cid == 0)
    def _():
        state_s[...] = init_ref[...]

    stim = steps_ref[:, 0:S]
    noise = steps_ref[:, S:S + R]

    for i in range(n_delay - 1):
        lo = (n_delay - 2 - i) * S
        row = state_s[:, lo:lo + S]
        e_s[i:i + 1, :] = row
        eb_s[i:i + 1, :] = row.astype(bf16)
    e_s[n_delay - 1:n_delay - 1 + CH, :] = stim
    eb_s[n_delay - 1:n_delay - 1 + CH, :] = stim.astype(bf16)
    d_mat_b = jnp.concatenate(
        [eb_s[n_delay - 1 - d:n_delay - 1 - d + CH, :] for d in range(n_delay)],
        axis=1)

    carry = state_s[:, DS:2 * DS]
    du = jnp.dot(d_mat_b, u2d_ref[...], preferred_element_type=f32)
    cu = jnp.dot(carry.astype(bf16), u2d_ref[...],
                 preferred_element_type=f32)
    temporal = (jnp.dot(l_ref[...], du, preferred_element_type=f32)
                + lam_ref[...] * cu)
    erows = (jnp.dot(lrows_ref[...], d_mat_b, preferred_element_type=f32)
             + jnp.concatenate(
                 [lam_ref[CH - 1:CH, :], lam_ref[last_local:last_local + 1, :],
                  jnp.zeros((6, 1), f32)], axis=0) * carry)

    fc1_b = b_ref[:, 0:H]
    fc2_b = b_ref[:, H:H + RO]

    res1 = jnp.dot(stim, w1_ref[...], preferred_element_type=f32)
    ws_out = res1[:, 0:RO]
    wc_out = res1[:, RO:RO + R]
    h = jnp.maximum(res1[:, RO + R:RO + R + H] + fc1_b, 0.0)

    valence = jnp.dot(h, w2_ref[...], preferred_element_type=f32) + fc2_b
    ro_pred = ws_out * valence

    lhs3 = jnp.concatenate([ro_pred, jnp.maximum(ro_pred, 0.0)], axis=1)
    res3 = jnp.dot(lhs3, w3_ref[...], preferred_element_type=f32)
    proactive = -res3[:, 0:R]
    reactive = jnp.maximum(res3[:, R:2 * R], 0.0)
    control = phi * (proactive + reactive)
    excitation = rho * wc_out

    bdt = beta * dt
    r_vec = bdt * (excitation + noise - 0.05 * control)
    sm1 = 1.0 - bdt * (excitation + control)

    c0 = state_s[:, 2 * DS:2 * DS + R]
    c_s[0:1, :] = c0
    c_s[1:CH + 1, :] = jnp.broadcast_to(c0, (CH, R))
    w_i = wi_ref[...]
    inh_q = []
    for m in range(_NSWEEP):
        src = c_s[0:CH, :]
        if m <= _NSWEEP - 1 - _LAG or m == 0:
            inh_q.append(jnp.dot(src, w_i, preferred_element_type=f32))
        inh = inh_q[max(0, m - _LAG)]
        pre = src * sm1 + r_vec - (src + 0.05) * inh
        c_s[1:CH + 1, :] = 0.5 * jnp.tanh(0.5 * pre) + 0.5
    c_all = c_s[1:CH + 1, :]

    out_ref[:, 0:R] = c_all
    out_ref[:, R:2 * R] = (c_all > response_threshold).astype(f32)
    out_ref[:, 2 * R:2 * R + RO] = ro_pred
    out_ref[:, 2 * R + RO:2 * R + 2 * RO] = temporal
    out_ref[:, 2 * R + 2 * RO:] = jnp.zeros(
        (CH, out_ref.shape[1] - (2 * R + 2 * RO)), f32)

    d_last = jnp.concatenate(
        [e_s[n_delay - 1 - d + CH - 1:n_delay - d + CH - 1, :]
         for d in range(n_delay)], axis=1)
    state_s[:, 0:DS] = d_last
    state_s[:, DS:2 * DS] = erows[0:1, :]
    state_s[:, 2 * DS:2 * DS + R] = c_s[CH:CH + 1, :]

    @pl.when(cid == last_chunk)
    def _():
        state_out_ref[:, 0:DS] = jnp.concatenate(
            [e_s[n_delay - 1 - d + last_local:n_delay - d + last_local, :]
             for d in range(n_delay)], axis=1)
        state_out_ref[:, DS:2 * DS] = erows[1:2, :]
        state_out_ref[:, 2 * DS:2 * DS + R] = c_s[last_local + 1:last_local + 2, :]


def kernel(W_S, W_C, W_F, W_R, W_I, U, fc1_w, fc1_b, fc2_w, fc2_b,
           delay_chain, eligibility_trace, C, stimuli_seq, noise_seq):
    f32 = jnp.float32
    bf16 = jnp.bfloat16
    hp = dict(dt=0.1, beta=0.1, lambda_decay=0.95, psi=0.1, phi=0.1, rho=0.1,
              response_threshold=0.5)
    n_ro, n_stim = W_S.shape
    n_resp = W_C.shape[0]
    n_delay = delay_chain.shape[0]
    hidden = fc1_b.shape[0]
    S, R, RO, H = n_stim, n_resp, n_ro, hidden
    DS = n_delay * S
    state_len = 2 * DS + R
    T = int(stimuli_seq.shape[0])
    OUT_W = 128
    assert 2 * R + 2 * RO <= OUT_W

    max_chunk = 256
    chunk = min(max_chunk, ((T + 7) // 8) * 8)
    t_pad = ((T + chunk - 1) // chunk) * chunk
    num_chunks = t_pad // chunk
    last_chunk = (T - 1) // chunk
    last_local = (T - 1) % chunk

    lam = float(hp["lambda_decay"])
    idx = jnp.arange(chunk)
    diff = idx[:, None] - idx[None, :]
    lam_mat = jnp.where(
        diff >= 0,
        jnp.power(jnp.float32(lam), jnp.maximum(diff, 0).astype(f32)),
        0.0).astype(f32)
    l_rows = jnp.zeros((8, chunk), f32)
    l_rows = l_rows.at[0].set(lam_mat[chunk - 1])
    l_rows = l_rows.at[1].set(lam_mat[last_local])
    l_rows = l_rows.astype(bf16)
    lam_pows = jnp.power(jnp.float32(lam),
                         (idx + 1).astype(f32)).reshape(chunk, 1)

    w1 = jnp.zeros((S, RO + R + H), f32)
    w1 = w1.at[:, :RO].set(W_S.T.astype(f32))
    w1 = w1.at[:, RO:RO + R].set(W_C.T.astype(f32))
    w1 = w1.at[:, RO + R:].set(fc1_w.T.astype(f32))

    w2 = fc2_w.T.astype(f32)
    u2d = U.astype(f32).reshape(RO, DS).T.astype(bf16)

    w3 = jnp.zeros((2 * RO, 2 * R), f32)
    w3 = w3.at[:RO, :R].set(W_F.astype(f32))
    w3 = w3.at[RO:2 * RO, R:2 * R].set(W_R.T.astype(f32))

    bdt = float(hp["beta"]) * float(hp["dt"])
    w_i = (bdt * float(hp["psi"])) * W_I.astype(f32)
    bias = jnp.concatenate([fc1_b.astype(f32),
                            fc2_b.astype(f32)]).reshape(1, H + RO)

    stim = jnp.zeros((t_pad, S), f32).at[:T].set(
        stimuli_seq.reshape(T, S).astype(f32))
    noz = jnp.zeros((t_pad, R), f32).at[:T].set(
        noise_seq.reshape(T, R).astype(f32))
    steps = jnp.concatenate([stim, noz], axis=1)

    init_state = jnp.concatenate(
        [delay_chain.astype(f32).reshape(1, DS),
         eligibility_trace.astype(f32).reshape(1, DS),
         C.astype(f32).reshape(1, R)], axis=1)

    kernel_fn = functools.partial(
        _pro_chunk_kernel,
        n_stim=S, n_resp=R, n_ro=RO, n_delay=n_delay, hidden=H,
        chunk=chunk, last_chunk=last_chunk, last_local=last_local,
        dt=float(hp["dt"]), beta=float(hp["beta"]),
        psi=float(hp["psi"]), phi=float(hp["phi"]), rho=float(hp["rho"]),
        response_threshold=float(hp["response_threshold"]),
    )

    def const_spec(shape):
        return pl.BlockSpec(shape, lambda c: (0,) * len(shape))

    per_step_out, final_state = pl.pallas_call(
        kernel_fn,
        grid=(num_chunks,),
        in_specs=[
            pl.BlockSpec((chunk, S + R), lambda c: (c, 0)),
            const_spec((1, state_len)),
            const_spec((chunk, chunk)),
            const_spec((8, chunk)),
            const_spec((chunk, 1)),
            const_spec(tuple(w1.shape)),
            const_spec(tuple(w2.shape)),
            const_spec(tuple(u2d.shape)),
            const_spec(tuple(w3.shape)),
            const_spec(tuple(w_i.shape)),
            const_spec(tuple(bias.shape)),
        ],
        out_specs=(
            pl.BlockSpec((chunk, OUT_W), lambda c: (c, 0)),
            const_spec((1, state_len)),
        ),
        out_shape=(
            jax.ShapeDtypeStruct((t_pad, OUT_W), f32),
            jax.ShapeDtypeStruct((1, state_len), f32),
        ),
        scratch_shapes=[
            pltpu.VMEM((1, state_len), f32),
            pltpu.VMEM((chunk + n_delay - 1, S), f32),
            pltpu.VMEM((chunk + n_delay - 1, S), jnp.bfloat16),
            pltpu.VMEM((chunk + 8, R), f32),
        ],
        compiler_params=pltpu.CompilerParams(
            dimension_semantics=("arbitrary",)),
    )(steps, init_state, lam_mat, l_rows, lam_pows, w1, w2, u2d, w3, w_i, bias)

    rows = per_step_out[:T]
    resp = rows[:, 0:R]
    disc = rows[:, R:2 * R]
    ro_pred = rows[:, 2 * R:2 * R + RO]
    temporal = rows[:, 2 * R + RO:2 * R + 2 * RO]
    fs = final_state[0]
    new_state = dict(
        delay_chain=fs[0:DS].reshape(n_delay, S),
        eligibility_trace=fs[DS:2 * DS].reshape(n_delay, S),
        C=fs[2 * DS:2 * DS + R],
    )
    return resp, ro_pred, temporal, disc, new_state


# speedup vs baseline: 15.9585x; 1.0576x over previous
import functools

import jax
import jax.numpy as jnp
from jax import lax
from jax.experimental import pallas as pl
from jax.experimental.pallas import tpu as pltpu

_NSWEEP = 20
_LAG = 7


def _pro_chunk_kernel(
    steps_ref,
    init_ref,
    l_ref,
    lrows_ref,
    lam_ref,
    w1_ref,
    w2_ref,
    u2d_ref,
    w3_ref,
    wi_ref,
    b_ref,
    out_ref,
    state_out_ref,
    state_s,
    e_s,
    eb_s,
    c_s,
    *,
    n_stim, n_resp, n_ro, n_delay, hidden,
    chunk, last_chunk, last_local,
    dt, beta, psi, phi, rho, response_threshold,
):
    f32 = jnp.float32
    bf16 = jnp.bfloat16
    S, R, RO, H = n_stim, n_resp, n_ro, hidden
    DS = n_delay * S
    CH = chunk
    cid = pl.program_id(0)

    @pl.when(cid == 0)
    def _():
        state_s[...] = init_ref[...]

    stim = steps_ref[:, 0:S]
    noise = steps_ref[:, S:S + R]

    OFF = 16
    for i in range(n_delay - 1):
        lo = (n_delay - 2 - i) * S
        row = state_s[:, lo:lo + S]
        e_s[OFF - (n_delay - 1) + i:OFF - (n_delay - 1) + i + 1, :] = row
        eb_s[OFF - (n_delay - 1) + i:OFF - (n_delay - 1) + i + 1, :] = row.astype(bf16)
    e_s[OFF:OFF + CH, :] = stim
    eb_s[OFF:OFF + CH, :] = stim.astype(bf16)
    d_mat_b = jnp.concatenate(
        [eb_s[OFF - d:OFF - d + CH, :] for d in range(n_delay)],
        axis=1)

    carry = state_s[:, DS:2 * DS]
    du = jnp.dot(d_mat_b, u2d_ref[...], preferred_element_type=f32)
    cu = jnp.dot(carry.astype(bf16), u2d_ref[...],
                 preferred_element_type=f32)
    temporal = (jnp.dot(l_ref[...], du, preferred_element_type=f32)
                + lam_ref[...] * cu)
    erows = (jnp.dot(lrows_ref[...], d_mat_b, preferred_element_type=f32)
             + jnp.concatenate(
                 [lam_ref[CH - 1:CH, :], lam_ref[last_local:last_local + 1, :],
                  jnp.zeros((6, 1), f32)], axis=0) * carry)

    fc1_b = b_ref[:, 0:H]
    fc2_b = b_ref[:, H:H + RO]

    res1 = jnp.dot(stim, w1_ref[...], preferred_element_type=f32)
    ws_out = res1[:, 0:RO]
    wc_out = res1[:, RO:RO + R]
    h = jnp.maximum(res1[:, RO + R:RO + R + H] + fc1_b, 0.0)

    valence = jnp.dot(h, w2_ref[...], preferred_element_type=f32) + fc2_b
    ro_pred = ws_out * valence

    lhs3 = jnp.concatenate([ro_pred, jnp.maximum(ro_pred, 0.0)], axis=1)
    res3 = jnp.dot(lhs3, w3_ref[...], preferred_element_type=f32)
    proactive = -res3[:, 0:R]
    reactive = jnp.maximum(res3[:, R:2 * R], 0.0)
    control = phi * (proactive + reactive)
    excitation = rho * wc_out

    bdt = beta * dt
    r_vec = bdt * (excitation + noise - 0.05 * control)
    sm1 = 1.0 - bdt * (excitation + control)

    c0 = state_s[:, 2 * DS:2 * DS + R]
    c_s[0:1, :] = c0
    c_s[1:CH + 1, :] = jnp.broadcast_to(c0, (CH, R))
    w_i = wi_ref[...]
    inh_q = []
    for m in range(_NSWEEP):
        src = c_s[0:CH, :]
        if m <= _NSWEEP - 1 - _LAG or m == 0:
            inh_q.append(jnp.dot(src, w_i, preferred_element_type=f32))
        inh = inh_q[max(0, m - _LAG)]
        pre = src * sm1 + r_vec - (src + 0.05) * inh
        c_s[1:CH + 1, :] = 0.5 * jnp.tanh(0.5 * pre) + 0.5
    c_all = c_s[1:CH + 1, :]

    out_ref[:, 0:R] = c_all
    out_ref[:, R:2 * R] = (c_all > response_threshold).astype(f32)
    out_ref[:, 2 * R:2 * R + RO] = ro_pred
    out_ref[:, 2 * R + RO:2 * R + 2 * RO] = temporal
    out_ref[:, 2 * R + 2 * RO:] = jnp.zeros(
        (CH, out_ref.shape[1] - (2 * R + 2 * RO)), f32)

    d_last = jnp.concatenate(
        [e_s[OFF - d + CH - 1:OFF - d + CH, :]
         for d in range(n_delay)], axis=1)
    state_s[:, 0:DS] = d_last
    state_s[:, DS:2 * DS] = erows[0:1, :]
    state_s[:, 2 * DS:2 * DS + R] = c_s[CH:CH + 1, :]

    @pl.when(cid == last_chunk)
    def _():
        state_out_ref[:, 0:DS] = jnp.concatenate(
            [e_s[OFF - d + last_local:OFF - d + last_local + 1, :]
             for d in range(n_delay)], axis=1)
        state_out_ref[:, DS:2 * DS] = erows[1:2, :]
        state_out_ref[:, 2 * DS:2 * DS + R] = c_s[last_local + 1:last_local + 2, :]


def kernel(W_S, W_C, W_F, W_R, W_I, U, fc1_w, fc1_b, fc2_w, fc2_b,
           delay_chain, eligibility_trace, C, stimuli_seq, noise_seq):
    f32 = jnp.float32
    bf16 = jnp.bfloat16
    hp = dict(dt=0.1, beta=0.1, lambda_decay=0.95, psi=0.1, phi=0.1, rho=0.1,
              response_threshold=0.5)
    n_ro, n_stim = W_S.shape
    n_resp = W_C.shape[0]
    n_delay = delay_chain.shape[0]
    hidden = fc1_b.shape[0]
    S, R, RO, H = n_stim, n_resp, n_ro, hidden
    DS = n_delay * S
    state_len = 2 * DS + R
    T = int(stimuli_seq.shape[0])
    OUT_W = 128
    assert 2 * R + 2 * RO <= OUT_W

    max_chunk = 512
    chunk = min(max_chunk, ((T + 7) // 8) * 8)
    t_pad = ((T + chunk - 1) // chunk) * chunk
    num_chunks = t_pad // chunk
    last_chunk = (T - 1) // chunk
    last_local = (T - 1) % chunk

    lam = float(hp["lambda_decay"])
    idx = jnp.arange(chunk)
    diff = idx[:, None] - idx[None, :]
    lam_mat = jnp.where(
        diff >= 0,
        jnp.power(jnp.float32(lam), jnp.maximum(diff, 0).astype(f32)),
        0.0).astype(f32)
    l_rows = jnp.zeros((8, chunk), f32)
    l_rows = l_rows.at[0].set(lam_mat[chunk - 1])
    l_rows = l_rows.at[1].set(lam_mat[last_local])
    l_rows = l_rows.astype(bf16)
    lam_pows = jnp.power(jnp.float32(lam),
                         (idx + 1).astype(f32)).reshape(chunk, 1)

    w1 = jnp.zeros((S, RO + R + H), f32)
    w1 = w1.at[:, :RO].set(W_S.T.astype(f32))
    w1 = w1.at[:, RO:RO + R].set(W_C.T.astype(f32))
    w1 = w1.at[:, RO + R:].set(fc1_w.T.astype(f32))

    w2 = fc2_w.T.astype(f32)
    u2d = U.astype(f32).reshape(RO, DS).T.astype(bf16)

    w3 = jnp.zeros((2 * RO, 2 * R), f32)
    w3 = w3.at[:RO, :R].set(W_F.astype(f32))
    w3 = w3.at[RO:2 * RO, R:2 * R].set(W_R.T.astype(f32))

    bdt = float(hp["beta"]) * float(hp["dt"])
    w_i = (bdt * float(hp["psi"])) * W_I.astype(f32)
    bias = jnp.concatenate([fc1_b.astype(f32),
                            fc2_b.astype(f32)]).reshape(1, H + RO)

    stim = jnp.zeros((t_pad, S), f32).at[:T].set(
        stimuli_seq.reshape(T, S).astype(f32))
    noz = jnp.zeros((t_pad, R), f32).at[:T].set(
        noise_seq.reshape(T, R).astype(f32))
    steps = jnp.concatenate([stim, noz], axis=1)

    init_state = jnp.concatenate(
        [delay_chain.astype(f32).reshape(1, DS),
         eligibility_trace.astype(f32).reshape(1, DS),
         C.astype(f32).reshape(1, R)], axis=1)

    kernel_fn = functools.partial(
        _pro_chunk_kernel,
        n_stim=S, n_resp=R, n_ro=RO, n_delay=n_delay, hidden=H,
        chunk=chunk, last_chunk=last_chunk, last_local=last_local,
        dt=float(hp["dt"]), beta=float(hp["beta"]),
        psi=float(hp["psi"]), phi=float(hp["phi"]), rho=float(hp["rho"]),
        response_threshold=float(hp["response_threshold"]),
    )

    def const_spec(shape):
        return pl.BlockSpec(shape, lambda c: (0,) * len(shape))

    per_step_out, final_state = pl.pallas_call(
        kernel_fn,
        grid=(num_chunks,),
        in_specs=[
            pl.BlockSpec((chunk, S + R), lambda c: (c, 0)),
            const_spec((1, state_len)),
            const_spec((chunk, chunk)),
            const_spec((8, chunk)),
            const_spec((chunk, 1)),
            const_spec(tuple(w1.shape)),
            const_spec(tuple(w2.shape)),
            const_spec(tuple(u2d.shape)),
            const_spec(tuple(w3.shape)),
            const_spec(tuple(w_i.shape)),
            const_spec(tuple(bias.shape)),
        ],
        out_specs=(
            pl.BlockSpec((chunk, OUT_W), lambda c: (c, 0)),
            const_spec((1, state_len)),
        ),
        out_shape=(
            jax.ShapeDtypeStruct((t_pad, OUT_W), f32),
            jax.ShapeDtypeStruct((1, state_len), f32),
        ),
        scratch_shapes=[
            pltpu.VMEM((1, state_len), f32),
            pltpu.VMEM((chunk + 16, S), f32),
            pltpu.VMEM((chunk + 16, S), jnp.bfloat16),
            pltpu.VMEM((chunk + 8, R), f32),
        ],
        compiler_params=pltpu.CompilerParams(
            dimension_semantics=("arbitrary",)),
    )(steps, init_state, lam_mat, l_rows, lam_pows, w1, w2, u2d, w3, w_i, bias)

    rows = per_step_out[:T]
    resp = rows[:, 0:R]
    disc = rows[:, R:2 * R]
    ro_pred = rows[:, 2 * R:2 * R + RO]
    temporal = rows[:, 2 * R + RO:2 * R + 2 * RO]
    fs = final_state[0]
    new_state = dict(
        delay_chain=fs[0:DS].reshape(n_delay, S),
        eligibility_trace=fs[DS:2 * DS].reshape(n_delay, S),
        C=fs[2 * DS:2 * DS + R],
    )
    return resp, ro_pred, temporal, disc, new_state
